# Initial kernel scaffold; baseline (speedup 1.0000x reference)
import jax
import jax.numpy as jnp
from jax.experimental import pallas as pl


def kernel(protac_components, poi_x, poi_edge_index, poi_edge_type, e3_x, e3_edge_index, e3_edge_type, params):
    raise NotImplementedError("write your pallas kernel here")


# speedup vs baseline: 1.5619x; 1.5619x over previous
import functools

import jax
import jax.numpy as jnp
from jax import lax
from jax.experimental import pallas as pl
from jax.experimental.pallas import tpu as pltpu
from jax.experimental.pallas import tpu_sc as plsc

N = 10000
E = 160000
R = 7
SEG = N * R
SEG_PAD = SEG + 16
NTILES = 16
EPT = 10240
E_PAD = NTILES * EPT
NB = 80
BW = 128
ZR = 547
ZSTEP = SEG_PAD // NTILES
OPT = SEG // NTILES
NBUF = 4
BN_ROWS = 1000
EPS = 1e-5


@functools.lru_cache(maxsize=None)
def _sc_scatter(n_chunks, dinp, rows2):
    cc = n_chunks // 2
    mesh = plsc.VectorSubcoreMesh(core_axis_name="c", subcore_axis_name="s",
                                  num_cores=2, num_subcores=NTILES)

    def body(h2, idx4, seg3, zz, agg, idx_v, seg_v, gbuf, zbuf, acc, sem):
        c = lax.axis_index("c")
        s = lax.axis_index("s")
        pltpu.sync_copy(seg3.at[s], seg_v)
        pltpu.sync_copy(zz, zbuf)
        for j in range(cc):
            f = c * cc + j
            pltpu.sync_copy(idx4.at[f, s], idx_v)
            for z in range(8):
                pltpu.sync_copy(zbuf, acc.at[pl.ds(s * ZSTEP + z * ZR, ZR)])
            plsc.subcore_barrier()
            for k in range(NBUF):
                pltpu.async_copy(h2.at[idx_v.at[k]], gbuf.at[k], sem)

            def superstep(t, carry):
                b = t * NBUF
                for k in range(NBUF):
                    g = b + k
                    pltpu.make_async_copy(
                        h2.at[idx_v.at[g]], gbuf.at[k], sem).wait()
                    pltpu.sync_copy(gbuf.at[k], acc.at[seg_v.at[g]], add=True)
                    nxt = g + NBUF

                    @pl.when(nxt < NB)
                    def _():
                        pltpu.async_copy(h2.at[idx_v.at[nxt]], gbuf.at[k], sem)
                return carry

            lax.fori_loop(0, NB // NBUF, superstep, 0)
            plsc.subcore_barrier()
            pltpu.sync_copy(
                acc.at[pl.ds(s * OPT, OPT)],
                agg.at[pl.ds(s * OPT, OPT), pl.ds(f * 16, 16)])
            plsc.subcore_barrier()

    return pl.kernel(
        body,
        out_type=jax.ShapeDtypeStruct((SEG, dinp), jnp.float32),
        mesh=mesh,
        scratch_types=[
            pltpu.VMEM((NB, BW), jnp.int32),
            pltpu.VMEM((NB, BW), jnp.int32),
            pltpu.VMEM((NBUF, BW, 16), jnp.float32),
            pltpu.VMEM((ZR, 16), jnp.float32),
            pltpu.VMEM_SHARED((SEG_PAD, 16), jnp.float32),
            pltpu.SemaphoreType.DMA,
        ],
        compiler_params=pltpu.CompilerParams(use_tc_tiling_on_sc=False),
    )


@functools.lru_cache(maxsize=None)
def _mm_stats(kab, ka_blk, dh):
    def body(agg_ref, h_ref, w_ref, ws_ref, b_ref, out_ref, st_ref, acc):
        i = pl.program_id(0)
        kb = pl.program_id(1)

        @pl.when(kb == 0)
        def _():
            acc[...] = jnp.zeros_like(acc)

        @pl.when(kb < kab)
        def _():
            acc[...] += jnp.dot(agg_ref[...], w_ref[...],
                                preferred_element_type=jnp.float32)

        @pl.when(kb == kab)
        def _():
            r = acc[...] + jnp.dot(h_ref[...], ws_ref[...],
                                   preferred_element_type=jnp.float32) + b_ref[...]
            out_ref[...] = r

            @pl.when(i == 0)
            def _():
                st_ref[...] = jnp.zeros_like(st_ref)

            st_ref[...] += jnp.concatenate(
                [jnp.sum(r, axis=0, keepdims=True),
                 jnp.sum(r * r, axis=0, keepdims=True)], axis=0)

    grid = (N // BN_ROWS, kab + 1)
    return pl.pallas_call(
        body,
        grid=grid,
        in_specs=[
            pl.BlockSpec((BN_ROWS, ka_blk),
                         lambda i, kb: (i, jnp.minimum(kb, kab - 1))),
            pl.BlockSpec((BN_ROWS, dh), lambda i, kb: (i, 0)),
            pl.BlockSpec((ka_blk, 512),
                         lambda i, kb: (jnp.minimum(kb, kab - 1), 0)),
            pl.BlockSpec((dh, 512), lambda i, kb: (0, 0)),
            pl.BlockSpec((1, 512), lambda i, kb: (0, 0)),
        ],
        out_specs=[
            pl.BlockSpec((BN_ROWS, 512), lambda i, kb: (i, 0)),
            pl.BlockSpec((2, 512), lambda i, kb: (0, 0)),
        ],
        out_shape=[
            jax.ShapeDtypeStruct((N, 512), jnp.float32),
            jax.ShapeDtypeStruct((2, 512), jnp.float32),
        ],
        scratch_shapes=[pltpu.VMEM((BN_ROWS, 512), jnp.float32)],
    )


@functools.lru_cache(maxsize=None)
def _bn_apply(residual):
    def body(out_ref, st_ref, gb_ref, *rest):
        if residual:
            hp_ref, h_ref, rs_ref = rest
        else:
            h_ref, rs_ref = rest
        i = pl.program_id(0)
        mu = st_ref[0:1, :] * (1.0 / N)
        var = st_ref[1:2, :] * (1.0 / N) - mu * mu
        sc = gb_ref[0:1, :] * lax.rsqrt(var + EPS)
        sh = gb_ref[1:2, :] - mu * sc
        hv = jnp.maximum(out_ref[...] * sc + sh, 0.0)
        if residual:
            hv = hv + hp_ref[...]
        h_ref[...] = hv

        @pl.when(i == 0)
        def _():
            rs_ref[...] = jnp.zeros_like(rs_ref)

        rs_ref[...] += jnp.sum(hv, axis=0, keepdims=True)

    in_specs = [
        pl.BlockSpec((BN_ROWS, 512), lambda i: (i, 0)),
        pl.BlockSpec((2, 512), lambda i: (0, 0)),
        pl.BlockSpec((2, 512), lambda i: (0, 0)),
    ]
    if residual:
        in_specs.append(pl.BlockSpec((BN_ROWS, 512), lambda i: (i, 0)))
    return pl.pallas_call(
        body,
        grid=(N // BN_ROWS,),
        in_specs=in_specs,
        out_specs=[
            pl.BlockSpec((BN_ROWS, 512), lambda i: (i, 0)),
            pl.BlockSpec((1, 512), lambda i: (0, 0)),
        ],
        out_shape=[
            jax.ShapeDtypeStruct((N, 512), jnp.float32),
            jax.ShapeDtypeStruct((1, 512), jnp.float32),
        ],
    )


@functools.lru_cache(maxsize=None)
def _head():
    def body(v_ref, w1_ref, b1_ref, w2_ref, b2_ref, w3_ref, b3_ref,
             out_ref, acc):
        kb = pl.program_id(0)

        @pl.when(kb == 0)
        def _():
            acc[...] = jnp.zeros_like(acc)

        acc[...] += jnp.dot(v_ref[...], w1_ref[...],
                            preferred_element_type=jnp.float32)

        @pl.when(kb == 11)
        def _():
            t = acc[...] + b1_ref[...]
            t = jnp.where(t > 0, t, 0.01 * t)
            t = jnp.dot(t, w2_ref[...],
                        preferred_element_type=jnp.float32) + b2_ref[...]
            t = jnp.where(t > 0, t, 0.01 * t)
            out_ref[...] = jnp.sum(t * w3_ref[...], axis=1,
                                   keepdims=True) + b3_ref[...]

    return pl.pallas_call(
        body,
        grid=(12,),
        in_specs=[
            pl.BlockSpec((1, 1024), lambda kb: (0, kb)),
            pl.BlockSpec((1024, 1024), lambda kb: (kb, 0)),
            pl.BlockSpec((1, 1024), lambda kb: (0, 0)),
            pl.BlockSpec((1024, 1024), lambda kb: (0, 0)),
            pl.BlockSpec((1, 1024), lambda kb: (0, 0)),
            pl.BlockSpec((1, 1024), lambda kb: (0, 0)),
            pl.BlockSpec((1, 1), lambda kb: (0, 0)),
        ],
        out_specs=pl.BlockSpec((1, 1), lambda kb: (0, 0)),
        out_shape=jax.ShapeDtypeStruct((1, 1), jnp.float32),
        scratch_shapes=[pltpu.VMEM((1, 1024), jnp.float32)],
    )


def kernel(protac_components, poi_x, poi_edge_index, poi_edge_type,
           e3_x, e3_edge_index, e3_edge_type, params):
    layers = params["layers"]
    w0p = jnp.pad(layers[0]["W"].reshape(R, 21, 512),
                  ((0, 0), (0, 11), (0, 0))).reshape(R * 32, 512)
    ws0p = jnp.pad(layers[0]["Wself"], ((0, 11), (0, 0)))

    def run_graph(x, ei, et):
        src = ei[0]
        seg = ei[1] * R + et
        pad_seg = jnp.full((E_PAD - E,), SEG, jnp.int32)
        seg3 = jnp.concatenate([seg, pad_seg]).reshape(NTILES, NB, BW)
        srcp = jnp.concatenate([src, jnp.zeros((E_PAD - E,), jnp.int32)])
        idx4 = (srcp[None, :] * 32
                + jnp.arange(32, dtype=jnp.int32)[:, None]
                ).reshape(32, NTILES, NB, BW)
        idx40 = (srcp[None, :] * 2
                 + jnp.arange(2, dtype=jnp.int32)[:, None]
                 ).reshape(2, NTILES, NB, BW)
        zz = jnp.zeros((ZR, 16), jnp.float32)
        xp = jnp.pad(x, ((0, 0), (0, 11)))

        agg = _sc_scatter(2, 32, N * 2)(xp.reshape(N * 2, 16), idx40, seg3, zz)
        out, st = _mm_stats(1, R * 32, 32)(
            agg.reshape(N, R * 32), xp, w0p, ws0p,
            layers[0]["b"].reshape(1, 512))
        gb = jnp.stack([layers[0]["gamma"], layers[0]["beta"]])
        h, rs = _bn_apply(False)(out, st, gb)
        sums = [rs]
        for l in range(1, 6):
            lp = layers[l]
            agg = _sc_scatter(32, 512, N * 32)(
                h.reshape(N * 32, 16), idx4, seg3, zz)
            out, st = _mm_stats(7, 512, 512)(
                agg.reshape(N, R * 512), h, lp["W"], lp["Wself"],
                lp["b"].reshape(1, 512))
            gb = jnp.stack([lp["gamma"], lp["beta"]])
            h, rs = _bn_apply(True)(out, st, gb, h)
            sums.append(rs)
        return jnp.concatenate(sums, axis=1)

    v0 = run_graph(poi_x, poi_edge_index, poi_edge_type)
    v1 = run_graph(e3_x, e3_edge_index, e3_edge_type)
    pc = protac_components.reshape(1, 6144)
    v = jnp.concatenate([v0, v1, pc], axis=1)
    return _head()(v, params["fc1_W"], params["fc1_b"].reshape(1, 1024),
                   params["fc2_W"], params["fc2_b"].reshape(1, 1024),
                   params["fc3_W"].reshape(1, 1024),
                   params["fc3_b"].reshape(1, 1))


# speedup vs baseline: 1.5661x; 1.0026x over previous
import functools

import jax
import jax.numpy as jnp
from jax import lax
from jax.experimental import pallas as pl
from jax.experimental.pallas import tpu as pltpu
from jax.experimental.pallas import tpu_sc as plsc

N = 10000
E = 160000
R = 7
SEG = N * R
SEG_PAD = SEG + 16
NTILES = 16
EPT = 10240
E_PAD = NTILES * EPT
NB = 80
BW = 128
NRING = 8
LAG = 4
ZR = 547
ZSTEP = SEG_PAD // NTILES
OPT = SEG // NTILES
NBUF = 4
BN_ROWS = 1000
EPS = 1e-5


@functools.lru_cache(maxsize=None)
def _sc_scatter(n_chunks, dinp, rows2):
    cc = n_chunks // 2
    mesh = plsc.VectorSubcoreMesh(core_axis_name="c", subcore_axis_name="s",
                                  num_cores=2, num_subcores=NTILES)

    def body(h2, idx4, seg3, zz, agg, idx_v, seg_v, gbuf, zbuf, acc,
             gsem, ssem):
        c = lax.axis_index("c")
        s = lax.axis_index("s")
        pltpu.sync_copy(seg3.at[s], seg_v)
        pltpu.sync_copy(zz, zbuf)
        for j in range(cc):
            f = c * cc + j
            pltpu.sync_copy(idx4.at[f, s], idx_v)
            for z in range(8):
                pltpu.sync_copy(zbuf, acc.at[pl.ds(s * ZSTEP + z * ZR, ZR)])
            plsc.subcore_barrier()
            for k in range(LAG):
                pltpu.async_copy(h2.at[idx_v.at[k]], gbuf.at[k], gsem)

            def superstep(t, carry):
                for kk in range(NRING):
                    g = t * NRING + kk
                    pltpu.make_async_copy(
                        h2.at[idx_v.at[g]], gbuf.at[kk], gsem).wait()
                    pltpu.async_copy(gbuf.at[kk], acc.at[seg_v.at[g]], ssem,
                                     add=True)

                    @pl.when(g >= LAG)
                    def _():
                        pltpu.make_async_copy(
                            gbuf.at[(kk + LAG) % NRING],
                            acc.at[seg_v.at[g - LAG]], ssem).wait()

                    nxt = g + LAG

                    @pl.when(nxt < NB)
                    def _():
                        pltpu.async_copy(h2.at[idx_v.at[nxt]],
                                         gbuf.at[(kk + LAG) % NRING], gsem)
                return carry

            lax.fori_loop(0, NB // NRING, superstep, 0)
            for g in range(NB - LAG, NB):
                pltpu.make_async_copy(
                    gbuf.at[g % NRING], acc.at[seg_v.at[g]], ssem).wait()
            plsc.subcore_barrier()
            pltpu.sync_copy(
                acc.at[pl.ds(s * OPT, OPT)],
                agg.at[pl.ds(s * OPT, OPT), pl.ds(f * 16, 16)])
            plsc.subcore_barrier()

    return pl.kernel(
        body,
        out_type=jax.ShapeDtypeStruct((SEG, dinp), jnp.float32),
        mesh=mesh,
        scratch_types=[
            pltpu.VMEM((NB, BW), jnp.int32),
            pltpu.VMEM((NB, BW), jnp.int32),
            pltpu.VMEM((NRING, BW, 16), jnp.float32),
            pltpu.VMEM((ZR, 16), jnp.float32),
            pltpu.VMEM_SHARED((SEG_PAD, 16), jnp.float32),
            pltpu.SemaphoreType.DMA,
            pltpu.SemaphoreType.DMA,
        ],
        compiler_params=pltpu.CompilerParams(use_tc_tiling_on_sc=False),
    )


@functools.lru_cache(maxsize=None)
def _mm_stats(kab, ka_blk, dh):
    def body(agg_ref, h_ref, w_ref, ws_ref, b_ref, out_ref, st_ref, acc):
        i = pl.program_id(0)
        kb = pl.program_id(1)

        @pl.when(kb == 0)
        def _():
            acc[...] = jnp.zeros_like(acc)

        @pl.when(kb < kab)
        def _():
            acc[...] += jnp.dot(agg_ref[...], w_ref[...],
                                preferred_element_type=jnp.float32)

        @pl.when(kb == kab)
        def _():
            r = acc[...] + jnp.dot(h_ref[...], ws_ref[...],
                                   preferred_element_type=jnp.float32) + b_ref[...]
            out_ref[...] = r

            @pl.when(i == 0)
            def _():
                st_ref[...] = jnp.zeros_like(st_ref)

            st_ref[...] += jnp.concatenate(
                [jnp.sum(r, axis=0, keepdims=True),
                 jnp.sum(r * r, axis=0, keepdims=True)], axis=0)

    grid = (N // BN_ROWS, kab + 1)
    return pl.pallas_call(
        body,
        grid=grid,
        in_specs=[
            pl.BlockSpec((BN_ROWS, ka_blk),
                         lambda i, kb: (i, jnp.minimum(kb, kab - 1))),
            pl.BlockSpec((BN_ROWS, dh), lambda i, kb: (i, 0)),
            pl.BlockSpec((ka_blk, 512),
                         lambda i, kb: (jnp.minimum(kb, kab - 1), 0)),
            pl.BlockSpec((dh, 512), lambda i, kb: (0, 0)),
            pl.BlockSpec((1, 512), lambda i, kb: (0, 0)),
        ],
        out_specs=[
            pl.BlockSpec((BN_ROWS, 512), lambda i, kb: (i, 0)),
            pl.BlockSpec((2, 512), lambda i, kb: (0, 0)),
        ],
        out_shape=[
            jax.ShapeDtypeStruct((N, 512), jnp.float32),
            jax.ShapeDtypeStruct((2, 512), jnp.float32),
        ],
        scratch_shapes=[pltpu.VMEM((BN_ROWS, 512), jnp.float32)],
    )


@functools.lru_cache(maxsize=None)
def _bn_apply(residual):
    def body(out_ref, st_ref, gb_ref, *rest):
        if residual:
            hp_ref, h_ref, rs_ref = rest
        else:
            h_ref, rs_ref = rest
        i = pl.program_id(0)
        mu = st_ref[0:1, :] * (1.0 / N)
        var = st_ref[1:2, :] * (1.0 / N) - mu * mu
        sc = gb_ref[0:1, :] * lax.rsqrt(var + EPS)
        sh = gb_ref[1:2, :] - mu * sc
        hv = jnp.maximum(out_ref[...] * sc + sh, 0.0)
        if residual:
            hv = hv + hp_ref[...]
        h_ref[...] = hv

        @pl.when(i == 0)
        def _():
            rs_ref[...] = jnp.zeros_like(rs_ref)

        rs_ref[...] += jnp.sum(hv, axis=0, keepdims=True)

    in_specs = [
        pl.BlockSpec((BN_ROWS, 512), lambda i: (i, 0)),
        pl.BlockSpec((2, 512), lambda i: (0, 0)),
        pl.BlockSpec((2, 512), lambda i: (0, 0)),
    ]
    if residual:
        in_specs.append(pl.BlockSpec((BN_ROWS, 512), lambda i: (i, 0)))
    return pl.pallas_call(
        body,
        grid=(N // BN_ROWS,),
        in_specs=in_specs,
        out_specs=[
            pl.BlockSpec((BN_ROWS, 512), lambda i: (i, 0)),
            pl.BlockSpec((1, 512), lambda i: (0, 0)),
        ],
        out_shape=[
            jax.ShapeDtypeStruct((N, 512), jnp.float32),
            jax.ShapeDtypeStruct((1, 512), jnp.float32),
        ],
    )


@functools.lru_cache(maxsize=None)
def _head():
    def body(v_ref, w1_ref, b1_ref, w2_ref, b2_ref, w3_ref, b3_ref,
             out_ref, acc):
        kb = pl.program_id(0)

        @pl.when(kb == 0)
        def _():
            acc[...] = jnp.zeros_like(acc)

        acc[...] += jnp.dot(v_ref[...], w1_ref[...],
                            preferred_element_type=jnp.float32)

        @pl.when(kb == 11)
        def _():
            t = acc[...] + b1_ref[...]
            t = jnp.where(t > 0, t, 0.01 * t)
            t = jnp.dot(t, w2_ref[...],
                        preferred_element_type=jnp.float32) + b2_ref[...]
            t = jnp.where(t > 0, t, 0.01 * t)
            out_ref[...] = jnp.sum(t * w3_ref[...], axis=1,
                                   keepdims=True) + b3_ref[...]

    return pl.pallas_call(
        body,
        grid=(12,),
        in_specs=[
            pl.BlockSpec((1, 1024), lambda kb: (0, kb)),
            pl.BlockSpec((1024, 1024), lambda kb: (kb, 0)),
            pl.BlockSpec((1, 1024), lambda kb: (0, 0)),
            pl.BlockSpec((1024, 1024), lambda kb: (0, 0)),
            pl.BlockSpec((1, 1024), lambda kb: (0, 0)),
            pl.BlockSpec((1, 1024), lambda kb: (0, 0)),
            pl.BlockSpec((1, 1), lambda kb: (0, 0)),
        ],
        out_specs=pl.BlockSpec((1, 1), lambda kb: (0, 0)),
        out_shape=jax.ShapeDtypeStruct((1, 1), jnp.float32),
        scratch_shapes=[pltpu.VMEM((1, 1024), jnp.float32)],
    )


def kernel(protac_components, poi_x, poi_edge_index, poi_edge_type,
           e3_x, e3_edge_index, e3_edge_type, params):
    layers = params["layers"]
    w0p = jnp.pad(layers[0]["W"].reshape(R, 21, 512),
                  ((0, 0), (0, 11), (0, 0))).reshape(R * 32, 512)
    ws0p = jnp.pad(layers[0]["Wself"], ((0, 11), (0, 0)))

    def run_graph(x, ei, et):
        src = ei[0]
        seg = ei[1] * R + et
        pad_seg = jnp.full((E_PAD - E,), SEG, jnp.int32)
        seg3 = jnp.concatenate([seg, pad_seg]).reshape(NTILES, NB, BW)
        srcp = jnp.concatenate([src, jnp.zeros((E_PAD - E,), jnp.int32)])
        idx4 = (srcp[None, :] * 32
                + jnp.arange(32, dtype=jnp.int32)[:, None]
                ).reshape(32, NTILES, NB, BW)
        idx40 = (srcp[None, :] * 2
                 + jnp.arange(2, dtype=jnp.int32)[:, None]
                 ).reshape(2, NTILES, NB, BW)
        zz = jnp.zeros((ZR, 16), jnp.float32)
        xp = jnp.pad(x, ((0, 0), (0, 11)))

        agg = _sc_scatter(2, 32, N * 2)(xp.reshape(N * 2, 16), idx40, seg3, zz)
        out, st = _mm_stats(1, R * 32, 32)(
            agg.reshape(N, R * 32), xp, w0p, ws0p,
            layers[0]["b"].reshape(1, 512))
        gb = jnp.stack([layers[0]["gamma"], layers[0]["beta"]])
        h, rs = _bn_apply(False)(out, st, gb)
        sums = [rs]
        for l in range(1, 6):
            lp = layers[l]
            agg = _sc_scatter(32, 512, N * 32)(
                h.reshape(N * 32, 16), idx4, seg3, zz)
            out, st = _mm_stats(7, 512, 512)(
                agg.reshape(N, R * 512), h, lp["W"], lp["Wself"],
                lp["b"].reshape(1, 512))
            gb = jnp.stack([lp["gamma"], lp["beta"]])
            h, rs = _bn_apply(True)(out, st, gb, h)
            sums.append(rs)
        return jnp.concatenate(sums, axis=1)

    v0 = run_graph(poi_x, poi_edge_index, poi_edge_type)
    v1 = run_graph(e3_x, e3_edge_index, e3_edge_type)
    pc = protac_components.reshape(1, 6144)
    v = jnp.concatenate([v0, v1, pc], axis=1)
    return _head()(v, params["fc1_W"], params["fc1_b"].reshape(1, 1024),
                   params["fc2_W"], params["fc2_b"].reshape(1, 1024),
                   params["fc3_W"].reshape(1, 1024),
                   params["fc3_b"].reshape(1, 1))


# speedup vs baseline: 2.3682x; 1.5122x over previous
import functools

import jax
import jax.numpy as jnp
from jax import lax
from jax.experimental import pallas as pl
from jax.experimental.pallas import tpu as pltpu
from jax.experimental.pallas import tpu_sc as plsc

N = 10000
E = 160000
R = 7
SEG = N * R
SEG_PAD = SEG + 16
NTILES = 16
EPT = 10240
E_PAD = NTILES * EPT
NB = 80
BW = 128
NRING = 8
LAG = 4
ZR = 547
ZSTEP = SEG_PAD // NTILES
OPT = SEG // NTILES
NBUF = 4
BN_ROWS = 2000
EPS = 1e-5


@functools.lru_cache(maxsize=None)
def _sc_scatter(n_chunks, dinp, rows2):
    cc = n_chunks // 2
    mesh = plsc.VectorSubcoreMesh(core_axis_name="c", subcore_axis_name="s",
                                  num_cores=2, num_subcores=NTILES)

    def body(h2, idx4, seg3, zz, agg, idx_v, seg_v, gbuf, zbuf, acc,
             gsem, ssem):
        c = lax.axis_index("c")
        s = lax.axis_index("s")
        pltpu.sync_copy(seg3.at[s], seg_v)
        pltpu.sync_copy(zz, zbuf)
        for j in range(cc):
            f = c * cc + j
            pltpu.sync_copy(idx4.at[f, s], idx_v)
            for z in range(8):
                pltpu.sync_copy(zbuf, acc.at[pl.ds(s * ZSTEP + z * ZR, ZR)])
            plsc.subcore_barrier()
            for k in range(LAG):
                pltpu.async_copy(h2.at[idx_v.at[k]], gbuf.at[k], gsem)

            def superstep(t, carry):
                for kk in range(NRING):
                    g = t * NRING + kk
                    pltpu.make_async_copy(
                        h2.at[idx_v.at[g]], gbuf.at[kk], gsem).wait()
                    pltpu.async_copy(gbuf.at[kk], acc.at[seg_v.at[g]], ssem,
                                     add=True)

                    @pl.when(g >= LAG)
                    def _():
                        pltpu.make_async_copy(
                            gbuf.at[(kk + LAG) % NRING],
                            acc.at[seg_v.at[g - LAG]], ssem).wait()

                    nxt = g + LAG

                    @pl.when(nxt < NB)
                    def _():
                        pltpu.async_copy(h2.at[idx_v.at[nxt]],
                                         gbuf.at[(kk + LAG) % NRING], gsem)
                return carry

            lax.fori_loop(0, NB // NRING, superstep, 0)
            for g in range(NB - LAG, NB):
                pltpu.make_async_copy(
                    gbuf.at[g % NRING], acc.at[seg_v.at[g]], ssem).wait()
            plsc.subcore_barrier()
            pltpu.sync_copy(
                acc.at[pl.ds(s * OPT, OPT)],
                agg.at[pl.ds(s * OPT, OPT), pl.ds(f * 32, 32)])
            plsc.subcore_barrier()

    return pl.kernel(
        body,
        out_type=jax.ShapeDtypeStruct((SEG, dinp), jnp.bfloat16),
        mesh=mesh,
        scratch_types=[
            pltpu.VMEM((NB, BW), jnp.int32),
            pltpu.VMEM((NB, BW), jnp.int32),
            pltpu.VMEM((NRING, BW, 32), jnp.bfloat16),
            pltpu.VMEM((ZR, 32), jnp.bfloat16),
            pltpu.VMEM_SHARED((SEG_PAD, 32), jnp.bfloat16),
            pltpu.SemaphoreType.DMA,
            pltpu.SemaphoreType.DMA,
        ],
        compiler_params=pltpu.CompilerParams(use_tc_tiling_on_sc=False),
    )


@functools.lru_cache(maxsize=None)
def _mm_stats(kab, ka_blk, dh):
    def body(agg_ref, h_ref, w_ref, ws_ref, b_ref, out_ref, st_ref, acc):
        i = pl.program_id(0)
        kb = pl.program_id(1)

        @pl.when(kb == 0)
        def _():
            acc[...] = jnp.zeros_like(acc)

        @pl.when(kb < kab)
        def _():
            acc[...] += jnp.dot(agg_ref[...].astype(jnp.float32), w_ref[...],
                                preferred_element_type=jnp.float32)

        @pl.when(kb == kab)
        def _():
            r = acc[...] + jnp.dot(h_ref[...].astype(jnp.float32), ws_ref[...],
                                   preferred_element_type=jnp.float32) + b_ref[...]
            out_ref[...] = r

            @pl.when(i == 0)
            def _():
                st_ref[...] = jnp.zeros_like(st_ref)

            st_ref[...] += jnp.concatenate(
                [jnp.sum(r, axis=0, keepdims=True),
                 jnp.sum(r * r, axis=0, keepdims=True)], axis=0)

    grid = (N // BN_ROWS, kab + 1)
    return pl.pallas_call(
        body,
        grid=grid,
        in_specs=[
            pl.BlockSpec((BN_ROWS, ka_blk),
                         lambda i, kb: (i, jnp.minimum(kb, kab - 1))),
            pl.BlockSpec((BN_ROWS, dh), lambda i, kb: (i, 0)),
            pl.BlockSpec((ka_blk, 512),
                         lambda i, kb: (jnp.minimum(kb, kab - 1), 0)),
            pl.BlockSpec((dh, 512), lambda i, kb: (0, 0)),
            pl.BlockSpec((1, 512), lambda i, kb: (0, 0)),
        ],
        out_specs=[
            pl.BlockSpec((BN_ROWS, 512), lambda i, kb: (i, 0)),
            pl.BlockSpec((2, 512), lambda i, kb: (0, 0)),
        ],
        out_shape=[
            jax.ShapeDtypeStruct((N, 512), jnp.float32),
            jax.ShapeDtypeStruct((2, 512), jnp.float32),
        ],
        scratch_shapes=[pltpu.VMEM((BN_ROWS, 512), jnp.float32)],
    )


@functools.lru_cache(maxsize=None)
def _bn_apply(residual):
    def body(out_ref, st_ref, gb_ref, *rest):
        if residual:
            hp_ref, h_ref, hb_ref, rs_ref = rest
        else:
            h_ref, hb_ref, rs_ref = rest
        i = pl.program_id(0)
        mu = st_ref[0:1, :] * (1.0 / N)
        var = st_ref[1:2, :] * (1.0 / N) - mu * mu
        sc = gb_ref[0:1, :] * lax.rsqrt(var + EPS)
        sh = gb_ref[1:2, :] - mu * sc
        hv = jnp.maximum(out_ref[...] * sc + sh, 0.0)
        if residual:
            hv = hv + hp_ref[...]
        h_ref[...] = hv
        hb_ref[...] = hv.astype(jnp.bfloat16)

        @pl.when(i == 0)
        def _():
            rs_ref[...] = jnp.zeros_like(rs_ref)

        rs_ref[...] += jnp.sum(hv, axis=0, keepdims=True)

    in_specs = [
        pl.BlockSpec((BN_ROWS, 512), lambda i: (i, 0)),
        pl.BlockSpec((2, 512), lambda i: (0, 0)),
        pl.BlockSpec((2, 512), lambda i: (0, 0)),
    ]
    if residual:
        in_specs.append(pl.BlockSpec((BN_ROWS, 512), lambda i: (i, 0)))
    return pl.pallas_call(
        body,
        grid=(N // BN_ROWS,),
        in_specs=in_specs,
        out_specs=[
            pl.BlockSpec((BN_ROWS, 512), lambda i: (i, 0)),
            pl.BlockSpec((BN_ROWS, 512), lambda i: (i, 0)),
            pl.BlockSpec((1, 512), lambda i: (0, 0)),
        ],
        out_shape=[
            jax.ShapeDtypeStruct((N, 512), jnp.float32),
            jax.ShapeDtypeStruct((N, 512), jnp.bfloat16),
            jax.ShapeDtypeStruct((1, 512), jnp.float32),
        ],
    )


@functools.lru_cache(maxsize=None)
def _head():
    def body(v_ref, w1_ref, b1_ref, w2_ref, b2_ref, w3_ref, b3_ref,
             out_ref, acc):
        kb = pl.program_id(0)

        @pl.when(kb == 0)
        def _():
            acc[...] = jnp.zeros_like(acc)

        acc[...] += jnp.dot(v_ref[...], w1_ref[...],
                            preferred_element_type=jnp.float32)

        @pl.when(kb == 11)
        def _():
            t = acc[...] + b1_ref[...]
            t = jnp.where(t > 0, t, 0.01 * t)
            t = jnp.dot(t, w2_ref[...],
                        preferred_element_type=jnp.float32) + b2_ref[...]
            t = jnp.where(t > 0, t, 0.01 * t)
            out_ref[...] = jnp.sum(t * w3_ref[...], axis=1,
                                   keepdims=True) + b3_ref[...]

    return pl.pallas_call(
        body,
        grid=(12,),
        in_specs=[
            pl.BlockSpec((1, 1024), lambda kb: (0, kb)),
            pl.BlockSpec((1024, 1024), lambda kb: (kb, 0)),
            pl.BlockSpec((1, 1024), lambda kb: (0, 0)),
            pl.BlockSpec((1024, 1024), lambda kb: (0, 0)),
            pl.BlockSpec((1, 1024), lambda kb: (0, 0)),
            pl.BlockSpec((1, 1024), lambda kb: (0, 0)),
            pl.BlockSpec((1, 1), lambda kb: (0, 0)),
        ],
        out_specs=pl.BlockSpec((1, 1), lambda kb: (0, 0)),
        out_shape=jax.ShapeDtypeStruct((1, 1), jnp.float32),
        scratch_shapes=[pltpu.VMEM((1, 1024), jnp.float32)],
    )


def kernel(protac_components, poi_x, poi_edge_index, poi_edge_type,
           e3_x, e3_edge_index, e3_edge_type, params):
    layers = params["layers"]
    w0p = jnp.pad(layers[0]["W"].reshape(R, 21, 512),
                  ((0, 0), (0, 43), (0, 0))).reshape(R * 64, 512)
    ws0p = jnp.pad(layers[0]["Wself"], ((0, 43), (0, 0)))

    zz = jnp.zeros((ZR, 32), jnp.bfloat16)

    def prep_graph(x, ei, et):
        src = ei[0]
        seg = ei[1] * R + et
        pad_seg = jnp.full((E_PAD - E,), SEG, jnp.int32)
        seg3 = jnp.concatenate([seg, pad_seg]).reshape(NTILES, NB, BW)
        srcp = jnp.concatenate([src, jnp.zeros((E_PAD - E,), jnp.int32)])
        idx4 = (srcp[None, :] * 16
                + jnp.arange(16, dtype=jnp.int32)[:, None]
                ).reshape(16, NTILES, NB, BW)
        idx40 = (srcp[None, :] * 2
                 + jnp.arange(2, dtype=jnp.int32)[:, None]
                 ).reshape(2, NTILES, NB, BW)
        xp = jnp.pad(x, ((0, 0), (0, 43)))
        return {"seg3": seg3, "idx4": idx4, "idx40": idx40, "h": xp,
                "hb": xp.astype(jnp.bfloat16), "sums": []}

    def layer_sc(g, l):
        if l == 0:
            return _sc_scatter(2, 64, N * 2)(
                g["hb"].reshape(N * 2, 32), g["idx40"], g["seg3"], zz)
        return _sc_scatter(16, 512, N * 16)(
            g["hb"].reshape(N * 16, 32), g["idx4"], g["seg3"], zz)

    def layer_tc(g, l, agg):
        lp = layers[l]
        gb = jnp.stack([lp["gamma"], lp["beta"]])
        if l == 0:
            out, st = _mm_stats(1, R * 64, 64)(
                agg.reshape(N, R * 64), g["h"], w0p, ws0p,
                lp["b"].reshape(1, 512))
            h, hb, rs = _bn_apply(False)(out, st, gb)
        else:
            out, st = _mm_stats(7, 512, 512)(
                agg.reshape(N, R * 512), g["h"], lp["W"], lp["Wself"],
                lp["b"].reshape(1, 512))
            h, hb, rs = _bn_apply(True)(out, st, gb, g["h"])
        g["h"] = h
        g["hb"] = hb
        g["sums"].append(rs)

    ga = prep_graph(poi_x, poi_edge_index, poi_edge_type)
    gb_ = prep_graph(e3_x, e3_edge_index, e3_edge_type)
    agg_a = layer_sc(ga, 0)
    for l in range(6):
        agg_b = layer_sc(gb_, l)
        layer_tc(ga, l, agg_a)
        if l < 5:
            agg_a = layer_sc(ga, l + 1)
        layer_tc(gb_, l, agg_b)
    v0 = jnp.concatenate(ga["sums"], axis=1)
    v1 = jnp.concatenate(gb_["sums"], axis=1)
    pc = protac_components.reshape(1, 6144)
    v = jnp.concatenate([v0, v1, pc], axis=1)
    return _head()(v, params["fc1_W"], params["fc1_b"].reshape(1, 1024),
                   params["fc2_W"], params["fc2_b"].reshape(1, 1024),
                   params["fc3_W"].reshape(1, 1024),
                   params["fc3_b"].reshape(1, 1))
